# baseline (speedup 1.0000x reference)
import jax
import jax.numpy as jnp
from jax import lax
from jax.experimental import pallas as pl
from jax.experimental.pallas import tpu as pltpu

N_IMG = 8
ROI_PER_IMG = 32
N_ROI = N_IMG * ROI_PER_IMG
C_FEAT = 128
CLASS_NUM = 80
POOL_H = 16
POOL_W = 16
HW = POOL_H * POOL_W
K_TOP = 100
NMS_THRESH = 0.5

RB = 8
NCHUNK = (CLASS_NUM * HW) // 128
NPAD = 4096
NBOX = ROI_PER_IMG * K_TOP
BLK = 256
BROW = BLK // 128


def _conv(x, w, b, pad):
    y = lax.conv_general_dilated(
        x, w, window_strides=(1, 1), padding=[(pad, pad), (pad, pad)],
        dimension_numbers=('NCHW', 'OIHW', 'NCHW'))
    return y + b[None, :, None, None]


def _head(x, params, name):
    h = jax.nn.relu(_conv(x, params[name + '_w1'], params[name + '_b1'], 1))
    return _conv(h, params[name + '_w2'], params[name + '_b2'], 0)


def _topk_body(hm_ref, ow_ref, rb_ref, sc_ref, lb_ref, x1_ref, y1_ref,
               x2_ref, y2_ref, okx_ref, oky_ref, wkx_ref, wky_ref, scr_ref):
    hm = hm_ref[...]
    pos = lax.broadcasted_iota(jnp.int32, (RB, CLASS_NUM, HW), 2)
    py = pos // POOL_W
    px = pos % POOL_W
    hmax = hm
    for dy in (-1, 0, 1):
        for dx in (-1, 0, 1):
            if dy == 0 and dx == 0:
                continue
            sh = dy * POOL_W + dx
            rolled = jnp.roll(hm, -sh, axis=2)
            okm = ((py + dy >= 0) & (py + dy < POOL_H)
                   & (px + dx >= 0) & (px + dx < POOL_W))
            hmax = jnp.maximum(hmax, jnp.where(okm, rolled, jnp.full_like(rolled, -jnp.inf)))
    s = jnp.where(hmax == hm, hm, jnp.zeros_like(hm))

    for c in range(CLASS_NUM):
        scr_ref[c, :, :] = s[:, c, 0:128]
        scr_ref[CLASS_NUM + c, :, :] = s[:, c, 128:256]
    mA = jnp.max(s[:, :, 0:128], axis=2)
    mB = jnp.max(s[:, :, 128:256], axis=2)
    m1 = jnp.concatenate([mA, mB], axis=1)

    ch_iota = lax.broadcasted_iota(jnp.int32, (1, NCHUNK), 1)
    fb_iota = (ch_iota % CLASS_NUM) * 256 + (ch_iota // CLASS_NUM) * 128
    ln_iota = lax.broadcasted_iota(jnp.int32, (1, 1, 128), 2)
    sub8 = lax.broadcasted_iota(jnp.int32, (RB, 128), 0)
    lane128 = lax.broadcasted_iota(jnp.int32, (RB, 128), 1)
    r_iota = lax.broadcasted_iota(jnp.int32, (RB, NCHUNK), 0)
    c_iota = lax.broadcasted_iota(jnp.int32, (RB, NCHUNK), 1)

    def step(k, carry):
        m1, sc_acc, idx_acc = carry
        for r in range(RB):
            m_r = jnp.max(m1[r:r + 1, :])
            fb_r = jnp.min(jnp.where(m1[r:r + 1, :] == m_r, fb_iota,
                                     jnp.full_like(fb_iota, 1 << 30)))
            c_r = (fb_r % 256) // 128 * CLASS_NUM + fb_r // 256
            row = scr_ref[pl.ds(c_r, 1), pl.ds(r, 1), :]
            p_r = jnp.min(jnp.where(row == m_r, ln_iota, jnp.full_like(ln_iota, 128)))
            fidx = fb_r + p_r
            newrow = jnp.where(ln_iota == p_r, jnp.full_like(row, -1.0), row)
            scr_ref[pl.ds(c_r, 1), pl.ds(r, 1), :] = newrow
            m1 = jnp.where((r_iota == r) & (c_iota == c_r),
                           jnp.full_like(m1, 0.0) + jnp.max(newrow), m1)
            here = (sub8 == r) & (lane128 == k)
            sc_acc = jnp.where(here, jnp.zeros_like(sc_acc) + m_r, sc_acc)
            idx_acc = jnp.where(here, jnp.zeros_like(idx_acc) + fidx, idx_acc)
        return m1, sc_acc, idx_acc

    sc0 = jnp.zeros((RB, 128), jnp.float32)
    ix0 = jnp.zeros((RB, 128), jnp.int32)
    _, sc_acc, idx_acc = lax.fori_loop(0, K_TOP, step, (m1, sc0, ix0))

    cat = idx_acc // HW
    spatial = idx_acc % HW
    ys = (spatial // POOL_W).astype(jnp.float32)
    xs = (spatial % POOL_W).astype(jnp.float32)

    ow = ow_ref[...]
    oh = (spatial[:, :, None]
          == lax.broadcasted_iota(jnp.int32, (RB, 128, HW), 2))

    def gath(ch):
        src = jnp.broadcast_to(ow[:, :, ch][:, None, :], (RB, 128, HW))
        return jnp.sum(jnp.where(oh, src, jnp.zeros_like(src)), axis=2)

    off_x, off_y, wh_x, wh_y = gath(0), gath(1), gath(2), gath(3)

    rbv = rb_ref[...]
    roi_w = rbv[:, 2:3] - rbv[:, 0:1]
    roi_h = rbv[:, 3:4] - rbv[:, 1:2]
    w_scale = roi_w / POOL_W
    h_scale = roi_h / POOL_H
    xs2 = (xs + off_x) * w_scale
    ys2 = (ys + off_y) * h_scale
    width = wh_x * w_scale / 4
    height = wh_y * h_scale / 4
    x1 = xs2 - width / 2 + rbv[:, 0:1]
    x2 = xs2 + width / 2 + rbv[:, 0:1]
    y1 = ys2 - height / 2 + rbv[:, 1:2]
    y2 = ys2 + height / 2 + rbv[:, 1:2]

    sc_ref[...] = sc_acc[:, :K_TOP]
    lb_ref[...] = cat[:, :K_TOP] + 1
    x1_ref[...] = x1[:, :K_TOP]
    y1_ref[...] = y1[:, :K_TOP]
    x2_ref[...] = x2[:, :K_TOP]
    y2_ref[...] = y2[:, :K_TOP]
    okx_ref[...] = off_x[:, :K_TOP]
    oky_ref[...] = off_y[:, :K_TOP]
    wkx_ref[...] = wh_x[:, :K_TOP]
    wky_ref[...] = wh_y[:, :K_TOP]


def _run_topk(hm_s, ow, roi_boxes):
    n = N_ROI // RB
    f32 = jnp.float32
    ospec = pl.BlockSpec((RB, K_TOP), lambda i: (i, 0))
    oshape_f = jax.ShapeDtypeStruct((N_ROI, K_TOP), f32)
    outs = pl.pallas_call(
        _topk_body,
        grid=(n,),
        in_specs=[
            pl.BlockSpec((RB, CLASS_NUM, HW), lambda i: (i, 0, 0)),
            pl.BlockSpec((RB, HW, 4), lambda i: (i, 0, 0)),
            pl.BlockSpec((RB, 4), lambda i: (i, 0)),
        ],
        out_specs=[ospec] * 10,
        out_shape=[oshape_f, jax.ShapeDtypeStruct((N_ROI, K_TOP), jnp.int32)]
        + [oshape_f] * 8,
        scratch_shapes=[pltpu.VMEM((NCHUNK, RB, 128), jnp.float32)],
    )(hm_s, ow, roi_boxes)
    return outs


def _bitonic(arrays, less):
    rows, cols = 32, 128
    riota = lax.broadcasted_iota(jnp.int32, (rows, cols), 0)
    ciota = lax.broadcasted_iota(jnp.int32, (rows, cols), 1)
    flat = riota * cols + ciota
    k = 2
    while k <= NPAD:
        j = k // 2
        while j >= 1:
            if j < cols:
                bit = (ciota & j) != 0
                pv = [jnp.where(bit, jnp.roll(a, j, axis=1),
                                jnp.roll(a, -j, axis=1)) for a in arrays]
            else:
                jr = j // cols
                bit = (riota & jr) != 0
                pv = [jnp.where(bit, jnp.roll(a, jr, axis=0),
                                jnp.roll(a, -jr, axis=0)) for a in arrays]
            t = less(pv, arrays)
            lo = (flat & j) == 0
            up = (flat & k) == 0
            take = t ^ (lo ^ up)
            arrays = [jnp.where(take, p, a) for p, a in zip(pv, arrays)]
            j //= 2
        k *= 2
    return arrays


def _rows_to_col(v2):
    eye = (lax.broadcasted_iota(jnp.int32, (128, 128), 0)
           == lax.broadcasted_iota(jnp.int32, (128, 128), 1))
    cols = []
    for r in range(BROW):
        row = v2[r:r + 1, :]
        if v2.dtype == jnp.bool_:
            cols.append(jnp.any(jnp.broadcast_to(row, (128, 128)) & eye, axis=1, keepdims=True))
        else:
            rowb = jnp.broadcast_to(row, (128, 128))
            cols.append(jnp.sum(jnp.where(eye, rowb, jnp.zeros_like(rowb)),
                                axis=1, keepdims=True))
    return jnp.concatenate(cols, axis=0)


def _nms_body(x1_ref, y1_ref, x2_ref, y2_ref, sc_ref, lb_ref, keep_ref):
    x1 = x1_ref[...].reshape(32, 128)
    y1 = y1_ref[...].reshape(32, 128)
    x2 = x2_ref[...].reshape(32, 128)
    y2 = y2_ref[...].reshape(32, 128)
    sc = sc_ref[...].reshape(32, 128)
    lab = lb_ref[...].reshape(32, 128).astype(jnp.float32)

    riota = lax.broadcasted_iota(jnp.int32, (32, 128), 0)
    ciota = lax.broadcasted_iota(jnp.int32, (32, 128), 1)
    flat = riota * 128 + ciota
    real = flat < NBOX

    big = jnp.float32(3.0e38)
    bmax = jnp.maximum(
        jnp.maximum(jnp.max(jnp.where(real, x1, jnp.full_like(x1, -big))),
                    jnp.max(jnp.where(real, y1, jnp.full_like(y1, -big)))),
        jnp.maximum(jnp.max(jnp.where(real, x2, jnp.full_like(x2, -big))),
                    jnp.max(jnp.where(real, y2, jnp.full_like(y2, -big)))))
    bmin = jnp.minimum(
        jnp.minimum(jnp.min(jnp.where(real, x1, jnp.full_like(x1, big))),
                    jnp.min(jnp.where(real, y1, jnp.full_like(y1, big)))),
        jnp.minimum(jnp.min(jnp.where(real, x2, jnp.full_like(x2, big))),
                    jnp.min(jnp.where(real, y2, jnp.full_like(y2, big)))))
    span = bmax - bmin + 1.0
    shift = lab * span
    arrs = [jnp.where(real, sc, jnp.full_like(sc, -big)), flat,
            x1 + shift, y1 + shift, x2 + shift, y2 + shift]

    def less_score(pv, mv):
        return (pv[0] > mv[0]) | ((pv[0] == mv[0]) & (pv[1] < mv[1]))

    _, idxs, a1, b1, a2, b2 = _bitonic(arrs, less_score)
    areas = jnp.maximum(a2 - a1, 0.0) * jnp.maximum(b2 - b1, 0.0)

    supp = jnp.zeros((32, 128), jnp.float32)
    keep_s = jnp.zeros((32, 128), jnp.float32)

    u_col = lax.broadcasted_iota(jnp.int32, (BLK, 1, 1), 0)
    t_flat2 = (lax.broadcasted_iota(jnp.int32, (BROW, 128), 0) * 128
               + lax.broadcasted_iota(jnp.int32, (BROW, 128), 1))
    lt_mask = (u_col < t_flat2[None, :, :]).astype(jnp.float32)

    for j in range(NPAD // BLK):
        r0 = j * BROW
        xb1 = _rows_to_col(a1[r0:r0 + BROW, :])[:, :, None]
        yb1 = _rows_to_col(b1[r0:r0 + BROW, :])[:, :, None]
        xb2 = _rows_to_col(a2[r0:r0 + BROW, :])[:, :, None]
        yb2 = _rows_to_col(b2[r0:r0 + BROW, :])[:, :, None]
        ab = _rows_to_col(areas[r0:r0 + BROW, :])[:, :, None]

        px1 = a1[r0:, :][None, :, :]
        py1 = b1[r0:, :][None, :, :]
        px2 = a2[r0:, :][None, :, :]
        py2 = b2[r0:, :][None, :, :]
        pa = areas[r0:, :][None, :, :]

        xx1 = jnp.maximum(xb1, px1)
        yy1 = jnp.maximum(yb1, py1)
        xx2 = jnp.minimum(xb2, px2)
        yy2 = jnp.minimum(yb2, py2)
        inter = jnp.maximum(xx2 - xx1, 0.0) * jnp.maximum(yy2 - yy1, 0.0)
        iou = inter / (ab + pa - inter + 1e-9)
        Mf = (iou > NMS_THRESH).astype(jnp.float32)

        Df = Mf[:, 0:BROW, :] * lt_mask
        supp_blk = supp[r0:r0 + BROW, :]

        def fix_cond(c):
            _, changed, it = c
            return changed & (it < BLK + 2)

        def fix_body(c):
            kept2, _, it = c
            kcol = _rows_to_col(kept2)[:, :, None]
            sfrom = jnp.max(Df * kcol, axis=0)
            kept_new = (1.0 - supp_blk) * (1.0 - sfrom)
            return kept_new, jnp.any(kept_new != kept2), it + 1

        kept2, _, _ = lax.while_loop(
            fix_cond, fix_body,
            (1.0 - supp_blk, jnp.bool_(True), jnp.int32(0)))

        pieces = [kept2]
        if r0 > 0:
            pieces.insert(0, jnp.zeros((r0, 128), jnp.float32))
        if 32 - r0 - BROW > 0:
            pieces.append(jnp.zeros((32 - r0 - BROW, 128), jnp.float32))
        blockmask = ((riota >= r0) & (riota < r0 + BROW)).astype(jnp.float32)
        keep_s = keep_s * (1.0 - blockmask) + jnp.concatenate(pieces, axis=0)

        kcolf = _rows_to_col(kept2)[:, :, None]
        contrib = jnp.max(Mf * kcolf, axis=0)
        if r0 > 0:
            contrib = jnp.concatenate(
                [jnp.zeros((r0, 128), jnp.float32), contrib], axis=0)
        supp = jnp.maximum(supp, contrib)

    keepf = keep_s

    def less_idx(pv, mv):
        return pv[0] < mv[0]

    _, keepu = _bitonic([idxs, keepf], less_idx)
    keep_ref[...] = (keepu > 0.5).reshape(1, 32, 128)


def _run_nms(x1, y1, x2, y2, sc, lb):
    pad2 = lambda v: jnp.concatenate(
        [v, jnp.zeros((N_IMG, NPAD - NBOX), v.dtype)], axis=1
    ).reshape(N_IMG, 32, 128)
    spec = pl.BlockSpec((1, 32, 128), lambda i: (i, 0, 0))
    keep = pl.pallas_call(
        _nms_body,
        grid=(N_IMG,),
        in_specs=[spec] * 6,
        out_specs=spec,
        out_shape=jax.ShapeDtypeStruct((N_IMG, 32, 128), jnp.bool_),
    )(pad2(x1), pad2(y1), pad2(x2), pad2(y2), pad2(sc), pad2(lb))
    return keep.reshape(N_IMG, NPAD)[:, 0:NBOX]


def kernel(roi_boxes, features, params, inputs, stride):
    heatmap = jax.nn.sigmoid(_head(features, params, 'heatmap'))
    offset = _head(features, params, 'offset')
    width_height = _head(features, params, 'width_height')

    hm_s = heatmap.reshape(N_ROI, CLASS_NUM, HW)
    ow = jnp.concatenate(
        [offset.reshape(N_ROI, 2, HW).transpose(0, 2, 1),
         width_height.reshape(N_ROI, 2, HW).transpose(0, 2, 1)], axis=2)

    rb = roi_boxes.reshape(-1, 4)
    (scores, labels, x1, y1, x2, y2,
     okx, oky, wkx, wky) = _run_topk(hm_s, ow, rb)
    offset_k = jnp.stack([okx, oky], axis=1)
    wh_k = jnp.stack([wkx, wky], axis=1)

    boxes_im = jnp.stack([x1, y1, x2, y2], axis=2).reshape(N_IMG, NBOX, 4)
    scores_im = scores.reshape(N_IMG, NBOX)
    labels_im = labels.reshape(N_IMG, NBOX)

    keep_mask = _run_nms(
        x1.reshape(N_IMG, NBOX), y1.reshape(N_IMG, NBOX),
        x2.reshape(N_IMG, NBOX), y2.reshape(N_IMG, NBOX),
        scores_im, labels_im)

    return (heatmap, offset_k, wh_k, boxes_im, scores_im, labels_im,
            keep_mask)


# speedup vs baseline: 8.6044x; 8.6044x over previous
import jax
import jax.numpy as jnp
from jax import lax
from jax.experimental import pallas as pl
from jax.experimental.pallas import tpu as pltpu

N_IMG = 8
ROI_PER_IMG = 32
N_ROI = N_IMG * ROI_PER_IMG
C_FEAT = 128
CLASS_NUM = 80
POOL_H = 16
POOL_W = 16
HW = POOL_H * POOL_W
K_TOP = 100
NMS_THRESH = 0.5

RB = 8
NCHUNK = (CLASS_NUM * HW) // 128
NPAD = 4096
NBOX = ROI_PER_IMG * K_TOP
BLK = 256
BROW = BLK // 128


def _conv(x, w, b, pad):
    y = lax.conv_general_dilated(
        x, w, window_strides=(1, 1), padding=[(pad, pad), (pad, pad)],
        dimension_numbers=('NCHW', 'OIHW', 'NCHW'))
    return y + b[None, :, None, None]


def _head(x, params, name):
    h = jax.nn.relu(_conv(x, params[name + '_w1'], params[name + '_b1'], 1))
    return _conv(h, params[name + '_w2'], params[name + '_b2'], 0)


LROI = 128


def _extract_body(hm_ref, sc_ref, ix_ref, scr_ref):
    hm = hm_ref[...]
    pos = lax.broadcasted_iota(jnp.int32, (CLASS_NUM, HW, LROI), 1)
    py = pos // POOL_W
    px = pos % POOL_W
    hmax = hm
    for dy in (-1, 0, 1):
        for dx in (-1, 0, 1):
            if dy == 0 and dx == 0:
                continue
            sh = dy * POOL_W + dx
            rolled = jnp.roll(hm, -sh, axis=1)
            okm = ((py + dy >= 0) & (py + dy < POOL_H)
                   & (px + dx >= 0) & (px + dx < POOL_W))
            hmax = jnp.maximum(
                hmax, jnp.where(okm, rolled, jnp.full_like(rolled, -jnp.inf)))
    s = jnp.where(hmax == hm, hm, jnp.zeros_like(hm))
    scr_ref[...] = s

    flat3 = (lax.broadcasted_iota(jnp.int32, (CLASS_NUM, HW, LROI), 0) * HW
             + lax.broadcasted_iota(jnp.int32, (CLASS_NUM, HW, LROI), 1))
    krow = lax.broadcasted_iota(jnp.int32, (128, LROI), 0)
    bigi = jnp.full_like(flat3, 1 << 30)

    def step(k, carry):
        sc_acc, ix_acc = carry
        X = scr_ref[...]
        m = jnp.max(X, axis=(0, 1), keepdims=True)
        cand = jnp.where(X == m, flat3, bigi)
        idx = jnp.min(cand, axis=(0, 1), keepdims=True)
        scr_ref[...] = jnp.where(flat3 == idx, jnp.full_like(X, -1.0), X)
        rowm = krow == k
        sc_acc = jnp.where(rowm, jnp.zeros_like(sc_acc) + m[0], sc_acc)
        ix_acc = jnp.where(rowm, jnp.zeros_like(ix_acc) + idx[0], ix_acc)
        return sc_acc, ix_acc

    sc0 = jnp.zeros((128, LROI), jnp.float32)
    ix0 = jnp.zeros((128, LROI), jnp.int32)
    sc_acc, ix_acc = lax.fori_loop(0, K_TOP, step, (sc0, ix0))
    sc_ref[...] = sc_acc
    ix_ref[...] = ix_acc


def _run_extract(hm_t):
    return pl.pallas_call(
        _extract_body,
        grid=(N_ROI // LROI,),
        in_specs=[pl.BlockSpec((CLASS_NUM, HW, LROI), lambda i: (0, 0, i))],
        out_specs=[pl.BlockSpec((128, LROI), lambda i: (0, i)),
                   pl.BlockSpec((128, LROI), lambda i: (0, i))],
        out_shape=[jax.ShapeDtypeStruct((128, N_ROI), jnp.float32),
                   jax.ShapeDtypeStruct((128, N_ROI), jnp.int32)],
        scratch_shapes=[pltpu.VMEM((CLASS_NUM, HW, LROI), jnp.float32)],
    )(hm_t)


def _decode_body(ix_ref, ow_ref, rb_ref, lb_ref, x1_ref, y1_ref,
                 x2_ref, y2_ref, okx_ref, oky_ref, wkx_ref, wky_ref):
    idx = ix_ref[...]
    cat = idx // HW
    spatial = idx % HW
    ys = (spatial // POOL_W).astype(jnp.float32)
    xs = (spatial % POOL_W).astype(jnp.float32)

    ow = ow_ref[...]
    oh = (spatial[:, :, None]
          == lax.broadcasted_iota(jnp.int32, (RB, K_TOP, HW), 2))

    def gath(ch):
        src = jnp.broadcast_to(ow[:, :, ch][:, None, :], (RB, K_TOP, HW))
        return jnp.sum(jnp.where(oh, src, jnp.zeros_like(src)), axis=2)

    off_x, off_y, wh_x, wh_y = gath(0), gath(1), gath(2), gath(3)

    rbv = rb_ref[...]
    w_scale = (rbv[:, 2:3] - rbv[:, 0:1]) / POOL_W
    h_scale = (rbv[:, 3:4] - rbv[:, 1:2]) / POOL_H
    xs2 = (xs + off_x) * w_scale
    ys2 = (ys + off_y) * h_scale
    width = wh_x * w_scale / 4
    height = wh_y * h_scale / 4
    lb_ref[...] = cat + 1
    x1_ref[...] = xs2 - width / 2 + rbv[:, 0:1]
    x2_ref[...] = xs2 + width / 2 + rbv[:, 0:1]
    y1_ref[...] = ys2 - height / 2 + rbv[:, 1:2]
    y2_ref[...] = ys2 + height / 2 + rbv[:, 1:2]
    okx_ref[...] = off_x
    oky_ref[...] = off_y
    wkx_ref[...] = wh_x
    wky_ref[...] = wh_y


def _run_decode(inds, ow, roi_boxes):
    ospec = pl.BlockSpec((RB, K_TOP), lambda i: (i, 0))
    oshape_f = jax.ShapeDtypeStruct((N_ROI, K_TOP), jnp.float32)
    return pl.pallas_call(
        _decode_body,
        grid=(N_ROI // RB,),
        in_specs=[
            pl.BlockSpec((RB, K_TOP), lambda i: (i, 0)),
            pl.BlockSpec((RB, HW, 4), lambda i: (i, 0, 0)),
            pl.BlockSpec((RB, 4), lambda i: (i, 0)),
        ],
        out_specs=[ospec] * 9,
        out_shape=[jax.ShapeDtypeStruct((N_ROI, K_TOP), jnp.int32)]
        + [oshape_f] * 8,
    )(inds, ow, roi_boxes)


def _bitonic(arrays, less):
    rows, cols = 32, 128
    riota = lax.broadcasted_iota(jnp.int32, (rows, cols), 0)
    ciota = lax.broadcasted_iota(jnp.int32, (rows, cols), 1)
    flat = riota * cols + ciota
    k = 2
    while k <= NPAD:
        j = k // 2
        while j >= 1:
            if j < cols:
                bit = (ciota & j) != 0
                pv = [jnp.where(bit, jnp.roll(a, j, axis=1),
                                jnp.roll(a, -j, axis=1)) for a in arrays]
            else:
                jr = j // cols
                bit = (riota & jr) != 0
                pv = [jnp.where(bit, jnp.roll(a, jr, axis=0),
                                jnp.roll(a, -jr, axis=0)) for a in arrays]
            t = less(pv, arrays)
            lo = (flat & j) == 0
            up = (flat & k) == 0
            take = t ^ (lo ^ up)
            arrays = [jnp.where(take, p, a) for p, a in zip(pv, arrays)]
            j //= 2
        k *= 2
    return arrays


def _rows_to_col(v2):
    eye = (lax.broadcasted_iota(jnp.int32, (128, 128), 0)
           == lax.broadcasted_iota(jnp.int32, (128, 128), 1))
    cols = []
    for r in range(BROW):
        row = v2[r:r + 1, :]
        if v2.dtype == jnp.bool_:
            cols.append(jnp.any(jnp.broadcast_to(row, (128, 128)) & eye, axis=1, keepdims=True))
        else:
            rowb = jnp.broadcast_to(row, (128, 128))
            cols.append(jnp.sum(jnp.where(eye, rowb, jnp.zeros_like(rowb)),
                                axis=1, keepdims=True))
    return jnp.concatenate(cols, axis=0)


def _nms_body(x1_ref, y1_ref, x2_ref, y2_ref, sc_ref, lb_ref, keep_ref):
    x1 = x1_ref[...].reshape(32, 128)
    y1 = y1_ref[...].reshape(32, 128)
    x2 = x2_ref[...].reshape(32, 128)
    y2 = y2_ref[...].reshape(32, 128)
    sc = sc_ref[...].reshape(32, 128)
    lab = lb_ref[...].reshape(32, 128).astype(jnp.float32)

    riota = lax.broadcasted_iota(jnp.int32, (32, 128), 0)
    ciota = lax.broadcasted_iota(jnp.int32, (32, 128), 1)
    flat = riota * 128 + ciota
    real = flat < NBOX

    big = jnp.float32(3.0e38)
    bmax = jnp.maximum(
        jnp.maximum(jnp.max(jnp.where(real, x1, jnp.full_like(x1, -big))),
                    jnp.max(jnp.where(real, y1, jnp.full_like(y1, -big)))),
        jnp.maximum(jnp.max(jnp.where(real, x2, jnp.full_like(x2, -big))),
                    jnp.max(jnp.where(real, y2, jnp.full_like(y2, -big)))))
    bmin = jnp.minimum(
        jnp.minimum(jnp.min(jnp.where(real, x1, jnp.full_like(x1, big))),
                    jnp.min(jnp.where(real, y1, jnp.full_like(y1, big)))),
        jnp.minimum(jnp.min(jnp.where(real, x2, jnp.full_like(x2, big))),
                    jnp.min(jnp.where(real, y2, jnp.full_like(y2, big)))))
    span = bmax - bmin + 1.0
    shift = lab * span
    arrs = [jnp.where(real, sc, jnp.full_like(sc, -big)), flat,
            x1 + shift, y1 + shift, x2 + shift, y2 + shift]

    def less_score(pv, mv):
        return (pv[0] > mv[0]) | ((pv[0] == mv[0]) & (pv[1] < mv[1]))

    _, idxs, a1, b1, a2, b2 = _bitonic(arrs, less_score)
    areas = jnp.maximum(a2 - a1, 0.0) * jnp.maximum(b2 - b1, 0.0)

    supp = jnp.zeros((32, 128), jnp.float32)
    keep_s = jnp.zeros((32, 128), jnp.float32)

    u_col = lax.broadcasted_iota(jnp.int32, (BLK, 1, 1), 0)
    t_flat2 = (lax.broadcasted_iota(jnp.int32, (BROW, 128), 0) * 128
               + lax.broadcasted_iota(jnp.int32, (BROW, 128), 1))
    lt_mask = (u_col < t_flat2[None, :, :]).astype(jnp.float32)

    for j in range(NPAD // BLK):
        r0 = j * BROW
        xb1 = _rows_to_col(a1[r0:r0 + BROW, :])[:, :, None]
        yb1 = _rows_to_col(b1[r0:r0 + BROW, :])[:, :, None]
        xb2 = _rows_to_col(a2[r0:r0 + BROW, :])[:, :, None]
        yb2 = _rows_to_col(b2[r0:r0 + BROW, :])[:, :, None]
        ab = _rows_to_col(areas[r0:r0 + BROW, :])[:, :, None]

        px1 = a1[r0:, :][None, :, :]
        py1 = b1[r0:, :][None, :, :]
        px2 = a2[r0:, :][None, :, :]
        py2 = b2[r0:, :][None, :, :]
        pa = areas[r0:, :][None, :, :]

        xx1 = jnp.maximum(xb1, px1)
        yy1 = jnp.maximum(yb1, py1)
        xx2 = jnp.minimum(xb2, px2)
        yy2 = jnp.minimum(yb2, py2)
        inter = jnp.maximum(xx2 - xx1, 0.0) * jnp.maximum(yy2 - yy1, 0.0)
        iou = inter / (ab + pa - inter + 1e-9)
        Mf = (iou > NMS_THRESH).astype(jnp.float32)

        Df = Mf[:, 0:BROW, :] * lt_mask
        supp_blk = supp[r0:r0 + BROW, :]

        def fix_cond(c):
            _, changed, it = c
            return changed & (it < BLK + 2)

        def fix_body(c):
            kept2, _, it = c
            kcol = _rows_to_col(kept2)[:, :, None]
            sfrom = jnp.max(Df * kcol, axis=0)
            kept_new = (1.0 - supp_blk) * (1.0 - sfrom)
            return kept_new, jnp.any(kept_new != kept2), it + 1

        kept2, _, _ = lax.while_loop(
            fix_cond, fix_body,
            (1.0 - supp_blk, jnp.bool_(True), jnp.int32(0)))

        pieces = [kept2]
        if r0 > 0:
            pieces.insert(0, jnp.zeros((r0, 128), jnp.float32))
        if 32 - r0 - BROW > 0:
            pieces.append(jnp.zeros((32 - r0 - BROW, 128), jnp.float32))
        blockmask = ((riota >= r0) & (riota < r0 + BROW)).astype(jnp.float32)
        keep_s = keep_s * (1.0 - blockmask) + jnp.concatenate(pieces, axis=0)

        kcolf = _rows_to_col(kept2)[:, :, None]
        contrib = jnp.max(Mf * kcolf, axis=0)
        if r0 > 0:
            contrib = jnp.concatenate(
                [jnp.zeros((r0, 128), jnp.float32), contrib], axis=0)
        supp = jnp.maximum(supp, contrib)

    keepf = keep_s

    def less_idx(pv, mv):
        return pv[0] < mv[0]

    _, keepu = _bitonic([idxs, keepf], less_idx)
    keep_ref[...] = (keepu > 0.5).reshape(1, 32, 128)


def _run_nms(x1, y1, x2, y2, sc, lb):
    pad2 = lambda v: jnp.concatenate(
        [v, jnp.zeros((N_IMG, NPAD - NBOX), v.dtype)], axis=1
    ).reshape(N_IMG, 32, 128)
    spec = pl.BlockSpec((1, 32, 128), lambda i: (i, 0, 0))
    keep = pl.pallas_call(
        _nms_body,
        grid=(N_IMG,),
        in_specs=[spec] * 6,
        out_specs=spec,
        out_shape=jax.ShapeDtypeStruct((N_IMG, 32, 128), jnp.bool_),
    )(pad2(x1), pad2(y1), pad2(x2), pad2(y2), pad2(sc), pad2(lb))
    return keep.reshape(N_IMG, NPAD)[:, 0:NBOX]


def kernel(roi_boxes, features, params, inputs, stride):
    heatmap = jax.nn.sigmoid(_head(features, params, 'heatmap'))
    offset = _head(features, params, 'offset')
    width_height = _head(features, params, 'width_height')

    hm_s = heatmap.reshape(N_ROI, CLASS_NUM, HW)
    ow = jnp.concatenate(
        [offset.reshape(N_ROI, 2, HW).transpose(0, 2, 1),
         width_height.reshape(N_ROI, 2, HW).transpose(0, 2, 1)], axis=2)

    rb = roi_boxes.reshape(-1, 4)
    hm_t = hm_s.transpose(1, 2, 0)
    sc_t, ix_t = _run_extract(hm_t)
    scores = sc_t[0:K_TOP, :].T
    inds = ix_t[0:K_TOP, :].T
    (labels, x1, y1, x2, y2,
     okx, oky, wkx, wky) = _run_decode(inds, ow, rb)
    offset_k = jnp.stack([okx, oky], axis=1)
    wh_k = jnp.stack([wkx, wky], axis=1)

    boxes_im = jnp.stack([x1, y1, x2, y2], axis=2).reshape(N_IMG, NBOX, 4)
    scores_im = scores.reshape(N_IMG, NBOX)
    labels_im = labels.reshape(N_IMG, NBOX)

    keep_mask = _run_nms(
        x1.reshape(N_IMG, NBOX), y1.reshape(N_IMG, NBOX),
        x2.reshape(N_IMG, NBOX), y2.reshape(N_IMG, NBOX),
        scores_im, labels_im)

    return (heatmap, offset_k, wh_k, boxes_im, scores_im, labels_im,
            keep_mask)
